# Initial kernel scaffold; baseline (speedup 1.0000x reference)
import jax
import jax.numpy as jnp
from jax.experimental import pallas as pl


def kernel(fwd_charIDs, bwd_charIDs, tags, C_emb, Wxf, Whf, bf, Wxb, Whb, bb, Wp, bp, Wwf, Uwf, bwf, Wwb, Uwb, bwb, Wo, bo, crf_w):
    raise NotImplementedError("write your pallas kernel here")


# speedup vs baseline: 17.1001x; 17.1001x over previous
import functools

import jax
import jax.numpy as jnp
from jax.experimental import pallas as pl
from jax.experimental.pallas import tpu as pltpu

T = 512
L = 16
NUM_CHARS = 256
NUM_TAGS = 1024
CE = 128
CR = 256
WR = 512
WE = 256
BEAM = 16

NEG_INF = float("-inf")


def _f32(x):
    return x.astype(jnp.float32)


def _dot(a, b):
    return jax.lax.dot_general(a, b, (((1,), (0,)), ((), ())),
                               preferred_element_type=jnp.float32)


def _row_to_col(row, n):
    eye = (jax.lax.broadcasted_iota(jnp.int32, (n, n), 0) ==
           jax.lax.broadcasted_iota(jnp.int32, (n, n), 1))
    big = jnp.broadcast_to(row, (n, n))
    zero = jnp.zeros((), row.dtype)
    return jnp.sum(jnp.where(eye, big, zero), axis=1, keepdims=True)


def _scores_kernel(idf_ref, idb_ref, cemb_ref, wxf_ref, whf_ref, bf_ref,
                   wxb_ref, whb_ref, bb_ref, wp_ref, bp_ref,
                   wwf_ref, uwf_ref, bwf_ref, wwb_ref, uwb_ref, bwb_ref,
                   wo_ref, bo_ref,
                   scores_ref,
                   af_ref, ab_ref, hfw_ref, hbw_ref):
    lane_c = jax.lax.broadcasted_iota(jnp.int32, (T, NUM_CHARS), 1)
    hf = jnp.zeros((T, CR), jnp.float32)
    hb = jnp.zeros((T, CR), jnp.float32)
    for t in range(L):
        ohf = jnp.where(idf_ref[:, t:t + 1] == lane_c, 1.0, 0.0)
        ohb = jnp.where(idb_ref[:, t:t + 1] == lane_c, 1.0, 0.0)
        xf = _dot(ohf, cemb_ref[...])
        xb = _dot(ohb, cemb_ref[...])
        hf = jnp.tanh(_dot(xf, wxf_ref[...]) + _dot(hf, whf_ref[...])
                      + bf_ref[...])
        hb = jnp.tanh(_dot(xb, wxb_ref[...]) + _dot(hb, whb_ref[...])
                      + bb_ref[...])
    cat = jnp.concatenate([hf, hb], axis=1)
    w_in = jnp.tanh(_dot(cat, wp_ref[...]) + bp_ref[...])
    af_ref[...] = _dot(w_in, wwf_ref[...]) + bwf_ref[...]
    ab_ref[...] = _dot(w_in, wwb_ref[...]) + bwb_ref[...]

    def word_step(t, carry):
        hfr, hbr = carry
        zf = _dot(hfr, uwf_ref[...])
        zb = _dot(hbr, uwb_ref[...])
        nhf = jnp.tanh(af_ref[pl.ds(t, 1), :] + zf)
        nhb = jnp.tanh(ab_ref[pl.ds(T - 1 - t, 1), :] + zb)
        hfw_ref[pl.ds(t, 1), :] = nhf
        hbw_ref[pl.ds(T - 1 - t, 1), :] = nhb
        return nhf, nhb

    z0 = jnp.zeros((1, WR), jnp.float32)
    jax.lax.fori_loop(0, T, word_step, (z0, z0))

    cat2 = jnp.concatenate([hfw_ref[...], hbw_ref[...]], axis=1)
    scores_ref[...] = _dot(cat2, wo_ref[...]) + bo_ref[...]


def _decode_kernel(scores_ref, crfw_ref, tags_ref,
                   pred_ref, nlp_ref,
                   s_ref, bs_ref, bt_ref, wall_ref, bps_ref, bi_ref):
    lane_t = jax.lax.broadcasted_iota(jnp.int32, (T, NUM_TAGS), 1)
    li16 = jax.lax.broadcasted_iota(jnp.int32, (1, BEAM), 1)
    si16 = jax.lax.broadcasted_iota(jnp.int32, (BEAM, BEAM), 0)

    s_ref[...] = scores_ref[...]
    for k in range(BEAM):
        s = s_ref[...]
        m = jnp.max(s, axis=1, keepdims=True)
        a = jnp.min(jnp.where(s == m, lane_t, NUM_TAGS), axis=1,
                    keepdims=True)
        bs_ref[:, k:k + 1] = m
        bt_ref[:, k:k + 1] = a
        s_ref[...] = jnp.where(lane_t == a, NEG_INF, s)

    def wchunk(g, carry):
        base = 7 * g
        chunk = bt_ref[pl.ds(base, 8), :]
        cols = [_row_to_col(chunk[k:k + 1, :], BEAM) for k in range(8)]
        col128 = jnp.concatenate(cols, axis=0)
        row128 = jnp.concatenate([chunk[k:k + 1, :] for k in range(8)],
                                 axis=1)
        oh = jnp.where(col128 == jax.lax.broadcasted_iota(
            jnp.int32, (128, NUM_TAGS), 1), 1.0, 0.0)
        oht = jnp.where(row128 == jax.lax.broadcasted_iota(
            jnp.int32, (NUM_TAGS, 128), 0), 1.0, 0.0)
        m1 = _dot(oh, crfw_ref[...])
        p = _dot(m1, oht)
        for k in range(7):
            wall_ref[pl.ds((base + k + 1) * BEAM, BEAM), :] = (
                p[BEAM * k:BEAM * (k + 1), BEAM * (k + 1):BEAM * (k + 2)])
        return carry

    jax.lax.fori_loop(0, 73, wchunk, 0)

    vs0 = bs_ref[0:1, :]

    def crf_step(t, carry):
        vs, fs = carry
        bs = bs_ref[pl.ds(t, 1), :]
        w = wall_ref[pl.ds(t * BEAM, BEAM), :]
        vs_col = _row_to_col(vs, BEAM)
        fs_col = _row_to_col(fs, BEAM)
        vv = vs_col + bs + w
        nvs = jnp.max(vv, axis=0, keepdims=True)
        bp = jnp.min(jnp.where(vv == nvs, si16, BEAM), axis=0, keepdims=True)
        bps_ref[pl.ds(t, 1), :] = bp
        vf = fs_col + bs + w
        off = jnp.max(vf, axis=0, keepdims=True)
        nfs = jnp.log(jnp.sum(jnp.exp(vf - off), axis=0, keepdims=True)) + off
        return nvs, nfs

    vs, fs = jax.lax.fori_loop(1, T, crf_step, (vs0, vs0))

    mv = jnp.max(vs)
    idx0 = jnp.min(jnp.where(vs == mv, li16, BEAM))
    bi_ref[pl.ds(T - 1, 1), :] = jnp.reshape(idx0, (1, 1))

    def back_step(u, idx):
        t = T - 1 - u
        row = bps_ref[pl.ds(t, 1), :]
        ni = jnp.sum(jnp.where(li16 == idx, row, 0))
        bi_ref[pl.ds(t - 1, 1), :] = jnp.reshape(ni, (1, 1))
        return ni

    jax.lax.fori_loop(0, T - 1, back_step, idx0)

    ohbi = bi_ref[...] == jax.lax.broadcasted_iota(jnp.int32, (T, BEAM), 1)
    pred_ref[...] = jnp.sum(jnp.where(ohbi, bt_ref[...], 0), axis=1,
                            keepdims=True)

    tag_oh = tags_ref[...] == lane_t
    base_s = jnp.sum(jnp.where(tag_oh, scores_ref[...], 0.0))

    si8 = jax.lax.broadcasted_iota(jnp.int32, (8, 8), 0)
    li8 = jax.lax.broadcasted_iota(jnp.int32, (8, 8), 1)
    superdiag = li8 == si8 + 1

    def tchunk(g, acc):
        chunk = tags_ref[pl.ds(7 * g, 8), :]
        oh = jnp.where(chunk == jax.lax.broadcasted_iota(
            jnp.int32, (8, NUM_TAGS), 1), 1.0, 0.0)
        row8r = jnp.sum(jnp.where(
            jax.lax.broadcasted_iota(jnp.int32, (8, 8), 0) ==
            jax.lax.broadcasted_iota(jnp.int32, (8, 8), 1),
            jnp.broadcast_to(chunk, (8, 8)), 0), axis=0, keepdims=True)
        oht = jnp.where(row8r == jax.lax.broadcasted_iota(
            jnp.int32, (NUM_TAGS, 8), 0), 1.0, 0.0)
        m = _dot(oh, crfw_ref[...])
        p = _dot(m, oht)
        return acc + jnp.sum(jnp.where(superdiag, p, 0.0))

    crf_s = jax.lax.fori_loop(0, 73, tchunk, jnp.float32(0.0))

    mf = jnp.max(fs)
    logz = jnp.log(jnp.sum(jnp.exp(fs - mf))) + mf
    nlp_ref[...] = jnp.reshape(logz - base_s - crf_s, (1, 1))


@jax.jit
def kernel(fwd_charIDs, bwd_charIDs, tags, C_emb, Wxf, Whf, bf, Wxb, Whb, bb,
           Wp, bp, Wwf, Uwf, bwf, Wwb, Uwb, bwb, Wo, bo, crf_w):
    scores = pl.pallas_call(
        _scores_kernel,
        out_shape=jax.ShapeDtypeStruct((T, NUM_TAGS), jnp.float32),
        scratch_shapes=[
            pltpu.VMEM((T, WR), jnp.float32),
            pltpu.VMEM((T, WR), jnp.float32),
            pltpu.VMEM((T, WR), jnp.float32),
            pltpu.VMEM((T, WR), jnp.float32),
        ],
    )(fwd_charIDs.astype(jnp.int32), bwd_charIDs.astype(jnp.int32),
      _f32(C_emb), _f32(Wxf), _f32(Whf), _f32(bf).reshape(1, CR),
      _f32(Wxb), _f32(Whb), _f32(bb).reshape(1, CR),
      _f32(Wp), _f32(bp).reshape(1, WE),
      _f32(Wwf), _f32(Uwf), _f32(bwf).reshape(1, WR),
      _f32(Wwb), _f32(Uwb), _f32(bwb).reshape(1, WR),
      _f32(Wo), _f32(bo).reshape(1, NUM_TAGS))

    pred, nlp = pl.pallas_call(
        _decode_kernel,
        out_shape=[
            jax.ShapeDtypeStruct((T, 1), jnp.int32),
            jax.ShapeDtypeStruct((1, 1), jnp.float32),
        ],
        scratch_shapes=[
            pltpu.VMEM((T, NUM_TAGS), jnp.float32),
            pltpu.VMEM((T, BEAM), jnp.float32),
            pltpu.VMEM((T, BEAM), jnp.int32),
            pltpu.VMEM(((T + 1) * BEAM, BEAM), jnp.float32),
            pltpu.VMEM((T, BEAM), jnp.int32),
            pltpu.VMEM((T, 1), jnp.int32),
        ],
    )(scores, _f32(crf_w), tags.astype(jnp.int32).reshape(T, 1))

    return pred.reshape(T), nlp[0, 0]


# speedup vs baseline: 22.6712x; 1.3258x over previous
import functools

import jax
import jax.numpy as jnp
from jax.experimental import pallas as pl
from jax.experimental.pallas import tpu as pltpu

T = 512
L = 16
NUM_CHARS = 256
NUM_TAGS = 1024
CE = 128
CR = 256
WR = 512
WE = 256
BEAM = 16

NEG_INF = float("-inf")


def _f32(x):
    return x.astype(jnp.float32)


def _dot(a, b):
    return jax.lax.dot_general(a, b, (((1,), (0,)), ((), ())),
                               preferred_element_type=jnp.float32)


def _row_to_col(row, n):
    eye = (jax.lax.broadcasted_iota(jnp.int32, (n, n), 0) ==
           jax.lax.broadcasted_iota(jnp.int32, (n, n), 1))
    big = jnp.broadcast_to(row, (n, n))
    zero = jnp.zeros((), row.dtype)
    return jnp.sum(jnp.where(eye, big, zero), axis=1, keepdims=True)


def _scores_kernel(idf_ref, idb_ref, cemb_ref, wxf_ref, whf_ref, bf_ref,
                   wxb_ref, whb_ref, bb_ref, wp_ref, bp_ref,
                   wwf_ref, uwf_ref, bwf_ref, wwb_ref, uwb_ref, bwb_ref,
                   wo_ref, bo_ref,
                   scores_ref,
                   af_ref, ab_ref, hfw_ref, hbw_ref):
    lane_c = jax.lax.broadcasted_iota(jnp.int32, (T, NUM_CHARS), 1)
    hf = jnp.zeros((T, CR), jnp.float32)
    hb = jnp.zeros((T, CR), jnp.float32)
    for t in range(L):
        ohf = jnp.where(idf_ref[:, t:t + 1] == lane_c, 1.0, 0.0)
        ohb = jnp.where(idb_ref[:, t:t + 1] == lane_c, 1.0, 0.0)
        xf = _dot(ohf, cemb_ref[...])
        xb = _dot(ohb, cemb_ref[...])
        hf = jnp.tanh(_dot(xf, wxf_ref[...]) + _dot(hf, whf_ref[...])
                      + bf_ref[...])
        hb = jnp.tanh(_dot(xb, wxb_ref[...]) + _dot(hb, whb_ref[...])
                      + bb_ref[...])
    cat = jnp.concatenate([hf, hb], axis=1)
    w_in = jnp.tanh(_dot(cat, wp_ref[...]) + bp_ref[...])
    af_ref[...] = _dot(w_in, wwf_ref[...]) + bwf_ref[...]
    ab_ref[...] = _dot(w_in, wwb_ref[...]) + bwb_ref[...]

    def word_step(t, carry):
        hfr, hbr = carry
        zf = _dot(hfr, uwf_ref[...])
        zb = _dot(hbr, uwb_ref[...])
        nhf = jnp.tanh(af_ref[pl.ds(t, 1), :] + zf)
        nhb = jnp.tanh(ab_ref[pl.ds(T - 1 - t, 1), :] + zb)
        hfw_ref[pl.ds(t, 1), :] = nhf
        hbw_ref[pl.ds(T - 1 - t, 1), :] = nhb
        return nhf, nhb

    z0 = jnp.zeros((1, WR), jnp.float32)
    jax.lax.fori_loop(0, T, word_step, (z0, z0), unroll=8)

    cat2 = jnp.concatenate([hfw_ref[...], hbw_ref[...]], axis=1)
    scores_ref[...] = _dot(cat2, wo_ref[...]) + bo_ref[...]


def _decode_kernel(scores_ref, crfw_ref, tags_ref,
                   pred_ref, nlp_ref,
                   s_ref, bs_ref, bt_ref, wall_ref, bps_ref):
    lane_t = jax.lax.broadcasted_iota(jnp.int32, (T, NUM_TAGS), 1)
    li16 = jax.lax.broadcasted_iota(jnp.int32, (1, BEAM), 1)
    si16 = jax.lax.broadcasted_iota(jnp.int32, (BEAM, BEAM), 0)

    s_ref[...] = scores_ref[...]
    for k in range(BEAM):
        s = s_ref[...]
        m = jnp.max(s, axis=1, keepdims=True)
        a = jnp.min(jnp.where(s == m, lane_t, NUM_TAGS), axis=1,
                    keepdims=True)
        bs_ref[:, k:k + 1] = m
        bt_ref[:, k:k + 1] = a
        s_ref[...] = jnp.where(lane_t == a, NEG_INF, s)

    def wchunk(g, carry):
        base = 7 * g
        chunk = bt_ref[pl.ds(base, 8), :]
        cols = [_row_to_col(chunk[k:k + 1, :], BEAM) for k in range(8)]
        col128 = jnp.concatenate(cols, axis=0)
        row128 = jnp.concatenate([chunk[k:k + 1, :] for k in range(8)],
                                 axis=1)
        oh = jnp.where(col128 == jax.lax.broadcasted_iota(
            jnp.int32, (128, NUM_TAGS), 1), 1.0, 0.0)
        oht = jnp.where(row128 == jax.lax.broadcasted_iota(
            jnp.int32, (NUM_TAGS, 128), 0), 1.0, 0.0)
        m1 = _dot(oh, crfw_ref[...])
        p = _dot(m1, oht)
        for k in range(7):
            wall_ref[pl.ds((base + k + 1) * BEAM, BEAM), :] = (
                p[BEAM * k:BEAM * (k + 1), BEAM * (k + 1):BEAM * (k + 2)])
        return carry

    jax.lax.fori_loop(0, 73, wchunk, 0)

    vs0 = bs_ref[0:1, :]

    def crf_step(t, carry):
        vs, fs = carry
        bs = bs_ref[pl.ds(t, 1), :]
        w = wall_ref[pl.ds(t * BEAM, BEAM), :]
        bw = bs + w
        vs_col = _row_to_col(vs, BEAM)
        fs_col = _row_to_col(fs, BEAM)
        vv = vs_col + bw
        nvs = jnp.max(vv, axis=0, keepdims=True)
        bp = jnp.min(jnp.where(vv == nvs, si16, BEAM), axis=0, keepdims=True)
        bps_ref[pl.ds(t, 1), :] = bp
        vf = fs_col + bw
        off = jnp.max(vf, axis=0, keepdims=True)
        nfs = jnp.log(jnp.sum(jnp.exp(vf - off), axis=0, keepdims=True)) + off
        return nvs, nfs

    vs, fs = jax.lax.fori_loop(1, T, crf_step, (vs0, vs0), unroll=4)

    mv = jnp.max(vs)
    idx0 = jnp.min(jnp.where(vs == mv, li16, BEAM))
    lane_b = jax.lax.broadcasted_iota(jnp.int32, (T, BEAM), 1)
    g = jnp.concatenate([bps_ref[pl.ds(1, T - 1), :], li16], axis=0)
    for r in range(9):
        s = 1 << r
        gs = jnp.concatenate(
            [g[s:, :], jnp.broadcast_to(li16, (s, BEAM)).astype(jnp.int32)],
            axis=0)
        acc = jnp.zeros((T, BEAM), jnp.int32)
        for j in range(BEAM):
            acc = acc + jnp.where(gs == j, g[:, j:j + 1], 0)
        g = acc
    bi = jnp.sum(jnp.where(lane_b == idx0, g, 0), axis=1, keepdims=True)
    ohbi = bi == lane_b
    pred_ref[...] = jnp.sum(jnp.where(ohbi, bt_ref[...], 0), axis=1,
                            keepdims=True)

    tag_oh = tags_ref[...] == lane_t
    base_s = jnp.sum(jnp.where(tag_oh, scores_ref[...], 0.0))

    si8 = jax.lax.broadcasted_iota(jnp.int32, (8, 8), 0)
    li8 = jax.lax.broadcasted_iota(jnp.int32, (8, 8), 1)
    superdiag = li8 == si8 + 1

    def tchunk(g, acc):
        chunk = tags_ref[pl.ds(7 * g, 8), :]
        oh = jnp.where(chunk == jax.lax.broadcasted_iota(
            jnp.int32, (8, NUM_TAGS), 1), 1.0, 0.0)
        row8r = jnp.sum(jnp.where(
            jax.lax.broadcasted_iota(jnp.int32, (8, 8), 0) ==
            jax.lax.broadcasted_iota(jnp.int32, (8, 8), 1),
            jnp.broadcast_to(chunk, (8, 8)), 0), axis=0, keepdims=True)
        oht = jnp.where(row8r == jax.lax.broadcasted_iota(
            jnp.int32, (NUM_TAGS, 8), 0), 1.0, 0.0)
        m = _dot(oh, crfw_ref[...])
        p = _dot(m, oht)
        return acc + jnp.sum(jnp.where(superdiag, p, 0.0))

    crf_s = jax.lax.fori_loop(0, 73, tchunk, jnp.float32(0.0))

    mf = jnp.max(fs)
    logz = jnp.log(jnp.sum(jnp.exp(fs - mf))) + mf
    nlp_ref[...] = jnp.reshape(logz - base_s - crf_s, (1, 1))


@jax.jit
def kernel(fwd_charIDs, bwd_charIDs, tags, C_emb, Wxf, Whf, bf, Wxb, Whb, bb,
           Wp, bp, Wwf, Uwf, bwf, Wwb, Uwb, bwb, Wo, bo, crf_w):
    scores = pl.pallas_call(
        _scores_kernel,
        out_shape=jax.ShapeDtypeStruct((T, NUM_TAGS), jnp.float32),
        scratch_shapes=[
            pltpu.VMEM((T, WR), jnp.float32),
            pltpu.VMEM((T, WR), jnp.float32),
            pltpu.VMEM((T, WR), jnp.float32),
            pltpu.VMEM((T, WR), jnp.float32),
        ],
    )(fwd_charIDs.astype(jnp.int32), bwd_charIDs.astype(jnp.int32),
      _f32(C_emb), _f32(Wxf), _f32(Whf), _f32(bf).reshape(1, CR),
      _f32(Wxb), _f32(Whb), _f32(bb).reshape(1, CR),
      _f32(Wp), _f32(bp).reshape(1, WE),
      _f32(Wwf), _f32(Uwf), _f32(bwf).reshape(1, WR),
      _f32(Wwb), _f32(Uwb), _f32(bwb).reshape(1, WR),
      _f32(Wo), _f32(bo).reshape(1, NUM_TAGS))

    pred, nlp = pl.pallas_call(
        _decode_kernel,
        out_shape=[
            jax.ShapeDtypeStruct((T, 1), jnp.int32),
            jax.ShapeDtypeStruct((1, 1), jnp.float32),
        ],
        scratch_shapes=[
            pltpu.VMEM((T, NUM_TAGS), jnp.float32),
            pltpu.VMEM((T, BEAM), jnp.float32),
            pltpu.VMEM((T, BEAM), jnp.int32),
            pltpu.VMEM(((T + 1) * BEAM, BEAM), jnp.float32),
            pltpu.VMEM((T, BEAM), jnp.int32),
        ],
    )(scores, _f32(crf_w), tags.astype(jnp.int32).reshape(T, 1))

    return pred.reshape(T), nlp[0, 0]


# speedup vs baseline: 29.1011x; 1.2836x over previous
import functools

import jax
import jax.numpy as jnp
from jax.experimental import pallas as pl
from jax.experimental.pallas import tpu as pltpu
from jax.experimental.pallas import tpu_sc as plsc

T = 512
L = 16
NUM_CHARS = 256
NUM_TAGS = 1024
CE = 128
CR = 256
WR = 512
WE = 256
BEAM = 16

NEG_INF = float("-inf")


def _f32(x):
    return x.astype(jnp.float32)


def _dot(a, b):
    return jax.lax.dot_general(a, b, (((1,), (0,)), ((), ())),
                               preferred_element_type=jnp.float32)


def _row_to_col(row, n):
    eye = (jax.lax.broadcasted_iota(jnp.int32, (n, n), 0) ==
           jax.lax.broadcasted_iota(jnp.int32, (n, n), 1))
    big = jnp.broadcast_to(row, (n, n))
    zero = jnp.zeros((), row.dtype)
    return jnp.sum(jnp.where(eye, big, zero), axis=1, keepdims=True)


def _scores_kernel(idf_ref, idb_ref, cemb_ref, wxf_ref, whf_ref, bf_ref,
                   wxb_ref, whb_ref, bb_ref, wp_ref, bp_ref,
                   wwf_ref, uwf_ref, bwf_ref, wwb_ref, uwb_ref, bwb_ref,
                   wo_ref, bo_ref,
                   scores_ref,
                   af_ref, ab_ref, hfw_ref, hbw_ref):
    lane_c = jax.lax.broadcasted_iota(jnp.int32, (T, NUM_CHARS), 1)
    hf = jnp.zeros((T, CR), jnp.float32)
    hb = jnp.zeros((T, CR), jnp.float32)
    for t in range(L):
        ohf = jnp.where(idf_ref[:, t:t + 1] == lane_c, 1.0, 0.0)
        ohb = jnp.where(idb_ref[:, t:t + 1] == lane_c, 1.0, 0.0)
        xf = _dot(ohf, cemb_ref[...])
        xb = _dot(ohb, cemb_ref[...])
        hf = jnp.tanh(_dot(xf, wxf_ref[...]) + _dot(hf, whf_ref[...])
                      + bf_ref[...])
        hb = jnp.tanh(_dot(xb, wxb_ref[...]) + _dot(hb, whb_ref[...])
                      + bb_ref[...])
    cat = jnp.concatenate([hf, hb], axis=1)
    w_in = jnp.tanh(_dot(cat, wp_ref[...]) + bp_ref[...])
    af_ref[...] = _dot(w_in, wwf_ref[...]) + bwf_ref[...]
    ab_ref[...] = _dot(w_in, wwb_ref[...]) + bwb_ref[...]

    def word_step(t, carry):
        hfr, hbr = carry
        zf = _dot(hfr, uwf_ref[...])
        zb = _dot(hbr, uwb_ref[...])
        nhf = jnp.tanh(af_ref[pl.ds(t, 1), :] + zf)
        nhb = jnp.tanh(ab_ref[pl.ds(T - 1 - t, 1), :] + zb)
        hfw_ref[pl.ds(t, 1), :] = nhf
        hbw_ref[pl.ds(T - 1 - t, 1), :] = nhb
        return nhf, nhb

    z0 = jnp.zeros((1, WR), jnp.float32)
    jax.lax.fori_loop(0, T, word_step, (z0, z0), unroll=8)

    cat2 = jnp.concatenate([hfw_ref[...], hbw_ref[...]], axis=1)
    scores_ref[...] = _dot(cat2, wo_ref[...]) + bo_ref[...]


def _sc_gather_kernel(btpad_hbm, crfw_hbm, tg_hbm, tgn_hbm,
                      wall_hbm, crfv_hbm,
                      btv, idxv, valsv, tgv, tgnv, tidxv, tvalsv, sem):
    c = jax.lax.axis_index("c")
    s = jax.lax.axis_index("s")
    wid = s * 2 + c

    pltpu.sync_copy(btpad_hbm.at[pl.ds(16 * wid, 24)], btv)

    def build(k, carry):
        prev = btv[k, :]
        cur = btv[k + 1, :]
        for i in range(16):
            idxv[pl.ds((16 * k + i) * 16, 16)] = prev[i] * NUM_TAGS + cur
        return carry

    jax.lax.fori_loop(0, 16, build, 0)

    copies = [
        pltpu.make_async_copy(crfw_hbm.at[idxv.at[pl.ds(128 * j, 128)]],
                              valsv.at[pl.ds(128 * j, 128)], sem)
        for j in range(32)
    ]
    for cp in copies:
        cp.start()
    for cp in copies:
        cp.wait()
    pltpu.sync_copy(valsv, wall_hbm.at[pl.ds(4096 * wid + 256, 4096)])

    @pl.when(wid == 0)
    def _():
        pltpu.sync_copy(tg_hbm, tgv)
        pltpu.sync_copy(tgn_hbm, tgnv)

        def tbuild(cc, carry):
            tidxv[pl.ds(16 * cc, 16)] = (tgv[pl.ds(16 * cc, 16)] * NUM_TAGS
                                         + tgnv[pl.ds(16 * cc, 16)])
            return carry

        jax.lax.fori_loop(0, 32, tbuild, 0)
        tcopies = [
            pltpu.make_async_copy(crfw_hbm.at[tidxv.at[pl.ds(128 * j, 128)]],
                                  tvalsv.at[pl.ds(128 * j, 128)], sem)
            for j in range(4)
        ]
        for cp in tcopies:
            cp.start()
        for cp in tcopies:
            cp.wait()
        pltpu.sync_copy(tvalsv, crfv_hbm)


def _sc_gather(btpad, crfw_flat, tg, tgn):
    mesh = plsc.VectorSubcoreMesh(core_axis_name="c", subcore_axis_name="s",
                                  num_cores=2, num_subcores=16)
    return pl.kernel(
        _sc_gather_kernel,
        out_type=[
            jax.ShapeDtypeStruct((131328,), jnp.float32),
            jax.ShapeDtypeStruct((T,), jnp.float32),
        ],
        mesh=mesh,
        scratch_types=[
            pltpu.VMEM((24, BEAM), jnp.int32),
            pltpu.VMEM((4096,), jnp.int32),
            pltpu.VMEM((4096,), jnp.float32),
            pltpu.VMEM((T,), jnp.int32),
            pltpu.VMEM((T,), jnp.int32),
            pltpu.VMEM((T,), jnp.int32),
            pltpu.VMEM((T,), jnp.float32),
            pltpu.SemaphoreType.DMA,
        ],
    )(btpad, crfw_flat, tg, tgn)


def _topk_kernel(scores_ref, tags_ref,
                 bs_ref, bt_ref, bsum_ref,
                 s_ref):
    lane_t = jax.lax.broadcasted_iota(jnp.int32, (T, NUM_TAGS), 1)

    s_ref[...] = scores_ref[...]
    for k in range(BEAM):
        s = s_ref[...]
        m = jnp.max(s, axis=1, keepdims=True)
        a = jnp.min(jnp.where(s == m, lane_t, NUM_TAGS), axis=1,
                    keepdims=True)
        bs_ref[:, k:k + 1] = m
        bt_ref[:, k:k + 1] = a
        s_ref[...] = jnp.where(lane_t == a, NEG_INF, s)

    tag_oh = tags_ref[...] == lane_t
    base_s = jnp.sum(jnp.where(tag_oh, scores_ref[...], 0.0))
    bsum_ref[...] = jnp.reshape(base_s, (1, 1))


def _viterbi_kernel(bs_ref, bt_ref, wall_ref, crfv_ref, bsum_ref,
                    pred_ref, nlp_ref,
                    bps_ref):
    li16 = jax.lax.broadcasted_iota(jnp.int32, (1, BEAM), 1)
    si16 = jax.lax.broadcasted_iota(jnp.int32, (BEAM, BEAM), 0)

    vs0 = bs_ref[0:1, :]

    def crf_step(t, carry):
        vs, fs = carry
        bs = bs_ref[pl.ds(t, 1), :]
        w = wall_ref[pl.ds(t * BEAM, BEAM), :]
        bw = bs + w
        vs_col = _row_to_col(vs, BEAM)
        fs_col = _row_to_col(fs, BEAM)
        vv = vs_col + bw
        nvs = jnp.max(vv, axis=0, keepdims=True)
        bp = jnp.min(jnp.where(vv == nvs, si16, BEAM), axis=0, keepdims=True)
        bps_ref[pl.ds(t, 1), :] = bp
        vf = fs_col + bw
        off = jnp.max(vf, axis=0, keepdims=True)
        nfs = jnp.log(jnp.sum(jnp.exp(vf - off), axis=0, keepdims=True)) + off
        return nvs, nfs

    vs, fs = jax.lax.fori_loop(1, T, crf_step, (vs0, vs0), unroll=4)

    mv = jnp.max(vs)
    idx0 = jnp.min(jnp.where(vs == mv, li16, BEAM))
    lane_b = jax.lax.broadcasted_iota(jnp.int32, (T, BEAM), 1)
    g = jnp.concatenate([bps_ref[pl.ds(1, T - 1), :], li16], axis=0)
    for r in range(9):
        s = 1 << r
        gs = jnp.concatenate(
            [g[s:, :], jnp.broadcast_to(li16, (s, BEAM)).astype(jnp.int32)],
            axis=0)
        acc = jnp.zeros((T, BEAM), jnp.int32)
        for j in range(BEAM):
            acc = acc + jnp.where(gs == j, g[:, j:j + 1], 0)
        g = acc
    bi = jnp.sum(jnp.where(lane_b == idx0, g, 0), axis=1, keepdims=True)
    ohbi = bi == lane_b
    pred_ref[...] = jnp.sum(jnp.where(ohbi, bt_ref[...], 0), axis=1,
                            keepdims=True)

    flat_pos = (jax.lax.broadcasted_iota(jnp.int32, (4, 128), 0) * 128 +
                jax.lax.broadcasted_iota(jnp.int32, (4, 128), 1))
    crf_s = jnp.sum(jnp.where(flat_pos < T - 1, crfv_ref[...], 0.0))

    mf = jnp.max(fs)
    logz = jnp.log(jnp.sum(jnp.exp(fs - mf))) + mf
    nlp_ref[...] = jnp.reshape(logz - bsum_ref[0, 0] - crf_s, (1, 1))


@jax.jit
def kernel(fwd_charIDs, bwd_charIDs, tags, C_emb, Wxf, Whf, bf, Wxb, Whb, bb,
           Wp, bp, Wwf, Uwf, bwf, Wwb, Uwb, bwb, Wo, bo, crf_w):
    scores = pl.pallas_call(
        _scores_kernel,
        out_shape=jax.ShapeDtypeStruct((T, NUM_TAGS), jnp.float32),
        scratch_shapes=[
            pltpu.VMEM((T, WR), jnp.float32),
            pltpu.VMEM((T, WR), jnp.float32),
            pltpu.VMEM((T, WR), jnp.float32),
            pltpu.VMEM((T, WR), jnp.float32),
        ],
    )(fwd_charIDs.astype(jnp.int32), bwd_charIDs.astype(jnp.int32),
      _f32(C_emb), _f32(Wxf), _f32(Whf), _f32(bf).reshape(1, CR),
      _f32(Wxb), _f32(Whb), _f32(bb).reshape(1, CR),
      _f32(Wp), _f32(bp).reshape(1, WE),
      _f32(Wwf), _f32(Uwf), _f32(bwf).reshape(1, WR),
      _f32(Wwb), _f32(Uwb), _f32(bwb).reshape(1, WR),
      _f32(Wo), _f32(bo).reshape(1, NUM_TAGS))

    tg = tags.astype(jnp.int32)
    bs, bt, bsum = pl.pallas_call(
        _topk_kernel,
        out_shape=[
            jax.ShapeDtypeStruct((T, BEAM), jnp.float32),
            jax.ShapeDtypeStruct((T, BEAM), jnp.int32),
            jax.ShapeDtypeStruct((1, 1), jnp.float32),
        ],
        scratch_shapes=[
            pltpu.VMEM((T, NUM_TAGS), jnp.float32),
        ],
    )(scores, tg.reshape(T, 1))

    btpad = jnp.pad(bt, ((0, 8), (0, 0)))
    tgn = jnp.concatenate([tg[1:], jnp.zeros((1,), jnp.int32)])
    wall_flat, crfv = _sc_gather(btpad, _f32(crf_w).reshape(-1), tg, tgn)

    pred, nlp = pl.pallas_call(
        _viterbi_kernel,
        out_shape=[
            jax.ShapeDtypeStruct((T, 1), jnp.int32),
            jax.ShapeDtypeStruct((1, 1), jnp.float32),
        ],
        scratch_shapes=[
            pltpu.VMEM((T, BEAM), jnp.int32),
        ],
    )(bs, bt, wall_flat.reshape((T + 1) * BEAM, BEAM),
      crfv.reshape(4, 128), bsum)

    return pred.reshape(T), nlp[0, 0]
